# Initial kernel scaffold; baseline (speedup 1.0000x reference)
import jax
import jax.numpy as jnp
from jax.experimental import pallas as pl


def kernel(type_ids, op_ids, fine_ids, obj_hashes, text_hashes, field_masks, time_feats, type_W, op_W, fine_W, obj_W, text_W, mask_W, mask_b, time_W, time_b, W1, b1, g1, bln1, W2, b2, g2, bln2):
    raise NotImplementedError("write your pallas kernel here")


# speedup vs baseline: 3.0189x; 3.0189x over previous
import functools

import jax
import jax.numpy as jnp
from jax import lax
from jax.experimental import pallas as pl
from jax.experimental.pallas import tpu as pltpu
from jax.experimental.pallas import tpu_sc as plsc

_NC = 2
_NS = 16
_NW = _NC * _NS
_CHUNK = 128

_F = 4
_SD = 64
_ED = 32


def _sc_gather_body(text_w, obj_w, idx_all, etext, eobj,
                    idx_v, rows_v, acc_v, obj_v, sem):
    n_rows = idx_all.shape[1]
    cpw = n_rows // 5
    c = lax.axis_index("c")
    s = lax.axis_index("s")
    w = s * _NC + c
    base0 = w * (cpw * _CHUNK)
    pltpu.sync_copy(idx_all.at[w], idx_v)

    def chunk_body(ci, _):
        base = base0 + ci * _CHUNK
        cps = [pltpu.async_copy(text_w.at[idx_v.at[f * cpw + ci]],
                                rows_v.at[f], sem)
               for f in range(_F)]
        cpo = pltpu.async_copy(obj_w.at[idx_v.at[_F * cpw + ci]], obj_v, sem)
        for cp in cps:
            cp.wait()
        cpo.wait()

        def tok_body(t, _):
            for k in range(_SD // 16):
                sl = pl.ds(k * 16, 16)
                acc_v[t, sl] = (rows_v[0, t, sl] + rows_v[1, t, sl]
                                + rows_v[2, t, sl] + rows_v[3, t, sl])
            return 0

        lax.fori_loop(0, _CHUNK, tok_body, 0)
        pltpu.sync_copy(acc_v, etext.at[pl.ds(base, _CHUNK)])
        pltpu.sync_copy(obj_v, eobj.at[pl.ds(base, _CHUNK)])
        return 0

    lax.fori_loop(0, cpw, chunk_body, 0)


def _sc_gather(text_w, obj_w, idx_all, n_tokens):
    mesh = plsc.VectorSubcoreMesh(core_axis_name="c", subcore_axis_name="s",
                                  num_cores=_NC, num_subcores=_NS)
    f = pl.kernel(
        _sc_gather_body,
        out_type=[
            jax.ShapeDtypeStruct((n_tokens, _SD), jnp.float32),
            jax.ShapeDtypeStruct((n_tokens, _ED), jnp.float32),
        ],
        mesh=mesh,
        scratch_types=[
            pltpu.VMEM(idx_all.shape[1:], jnp.int32),
            pltpu.VMEM((_F, _CHUNK, _SD), jnp.float32),
            pltpu.VMEM((_CHUNK, _SD), jnp.float32),
            pltpu.VMEM((_CHUNK, _ED), jnp.float32),
            pltpu.SemaphoreType.DMA,
        ],
        compiler_params=pltpu.CompilerParams(use_tc_tiling_on_sc=False),
    )
    return f(text_w, obj_w, idx_all)


def _tc_body(tids, oids, fids, masks, tfeat, etext, eobj,
             type_w, op_w, fine_w, mask_w, mask_b, time_w, time_b,
             w1, b1, g1, bl1, w2, b2, g2, bl2, out):
    blk = masks.shape[0]
    f32 = jnp.float32

    def onehot_emb(ids_ref, table_ref):
        ids = ids_ref[0, 0, :]
        nrow = table_ref.shape[0]
        oh = (lax.broadcasted_iota(jnp.int32, (blk, nrow), 1)
              == ids[:, None]).astype(f32)
        return jnp.dot(oh, table_ref[...], preferred_element_type=f32)

    e_type = onehot_emb(tids, type_w)
    e_op = onehot_emb(oids, op_w)
    e_fine = onehot_emb(fids, fine_w)
    e_mask = jnp.dot(masks[...], mask_w[...],
                     preferred_element_type=f32) + mask_b[...]
    e_time = jnp.dot(tfeat[...], time_w[...],
                     preferred_element_type=f32) + time_b[...]
    concat = jnp.concatenate(
        [e_type, e_op, e_fine, eobj[...], etext[...], e_mask, e_time], axis=-1)

    def ln(x, g, b):
        m = jnp.mean(x, axis=-1, keepdims=True)
        v = jnp.mean((x - m) * (x - m), axis=-1, keepdims=True)
        return (x - m) * lax.rsqrt(v + 1e-5) * g + b

    h = jnp.dot(concat, w1[...], preferred_element_type=f32) + b1[...]
    h = ln(h, g1[...], bl1[...])
    h = jnp.maximum(h, 0.0)
    h = jnp.dot(h, w2[...], preferred_element_type=f32) + b2[...]
    out[...] = ln(h, g2[...], bl2[...])


def _tc_mlp(tids3, oids3, fids3, masks, tfeat, etext, eobj, weights, tblk):
    n = masks.shape[0]
    grid = (n // tblk,)
    od = weights[-4].shape[1]

    def ids_spec():
        return pl.BlockSpec((1, 1, tblk), lambda i: (i, 0, 0))

    def row_spec(d):
        return pl.BlockSpec((tblk, d), lambda i: (i, 0))

    def full_spec(shape):
        nd = len(shape)
        return pl.BlockSpec(shape, lambda i: (0,) * nd)

    in_specs = [
        ids_spec(), ids_spec(), ids_spec(),
        row_spec(masks.shape[1]), row_spec(tfeat.shape[1]),
        row_spec(etext.shape[1]), row_spec(eobj.shape[1]),
    ] + [full_spec(w.shape) for w in weights]

    return pl.pallas_call(
        _tc_body,
        grid=grid,
        in_specs=in_specs,
        out_specs=pl.BlockSpec((tblk, od), lambda i: (i, 0)),
        out_shape=jax.ShapeDtypeStruct((n, od), jnp.float32),
    )(tids3, oids3, fids3, masks, tfeat, etext, eobj, *weights)


def kernel(type_ids, op_ids, fine_ids, obj_hashes, text_hashes, field_masks,
           time_feats, type_W, op_W, fine_W, obj_W, text_W, mask_W, mask_b,
           time_W, time_b, W1, b1, g1, bln1, W2, b2, g2, bln2):
    B, L = type_ids.shape
    N = B * L
    tpw = N // _NW
    cpw = tpw // _CHUNK

    th = text_hashes.reshape(N, _F).astype(jnp.int32)
    tidx = th.T.reshape(_F, _NW, cpw, _CHUNK).transpose(1, 0, 2, 3)
    oidx = obj_hashes.astype(jnp.int32).reshape(_NW, 1, cpw, _CHUNK)
    idx_all = jnp.concatenate([tidx, oidx], axis=1).reshape(
        _NW, 5 * cpw, _CHUNK)

    e_text, e_obj = _sc_gather(text_W, obj_W, idx_all, N)

    tblk = 1024
    g = N // tblk
    tids3 = type_ids.astype(jnp.int32).reshape(g, 1, tblk)
    oids3 = op_ids.astype(jnp.int32).reshape(g, 1, tblk)
    fids3 = fine_ids.astype(jnp.int32).reshape(g, 1, tblk)
    masks = field_masks.astype(jnp.float32).reshape(N, field_masks.shape[-1])
    tfeat = time_feats.reshape(N, time_feats.shape[-1])
    weights = (type_W, op_W, fine_W, mask_W, mask_b.reshape(1, -1),
               time_W, time_b.reshape(1, -1), W1, b1.reshape(1, -1),
               g1.reshape(1, -1), bln1.reshape(1, -1), W2, b2.reshape(1, -1),
               g2.reshape(1, -1), bln2.reshape(1, -1))

    out = _tc_mlp(tids3, oids3, fids3, masks, tfeat, e_text, e_obj,
                  weights, tblk)
    return out.reshape(B, L, -1)
